# baseline (speedup 1.0000x reference)
import functools

import jax
import jax.numpy as jnp
from jax import lax
from jax.experimental import pallas as pl
from jax.experimental.pallas import tpu as pltpu
from jax.experimental.pallas import tpu_sc as plsc

N = 10000
E = 320000
G = 64
H = 64
NF = 81
EF = 7
SEQ = 15 * 2560
D = 128
NSUB = 16
NCORE = 2
RPS = 1000
NSTRIPE = N // RPS
GW = 128
SW = 128
BE = 2560

_F32 = jnp.float32
_BF16 = jnp.bfloat16


def _silu(v):
    return v * (1.0 / (1.0 + jnp.exp(-v)))


def _dot(a, b):
    return jnp.dot(a, b, preferred_element_type=_F32)


def _dot_t(a, b):
    return lax.dot_general(a, b, (((1,), (1,)), ((), ())),
                           preferred_element_type=_F32)


def _prologue1_body(x_ref, coords_ref, bcol_ref, brow_ref, wemb_ref, bemb_ref,
                    wc_ref, bc_ref, h_ref):
    coords = coords_ref[...]
    S = (bcol_ref[...] ==
         lax.broadcasted_iota(jnp.int32, (N, G), 1)).astype(_F32)
    St = (brow_ref[...] ==
          lax.broadcasted_iota(jnp.int32, (G, N), 0)).astype(_F32)
    h = _dot(x_ref[...], wemb_ref[...]) + bemb_ref[...]
    csum = _dot(St, coords)
    cnt = jnp.sum(St, axis=1, keepdims=True)
    center = csum / jnp.maximum(cnt, 1.0)
    diff = coords - _dot(S, center)
    radius = jnp.sqrt(jnp.sum(diff * diff, axis=1, keepdims=True))
    h_ref[...] = h + _silu(radius * wc_ref[...] + bc_ref[...])


def _prologue1(x, coords, bcol, brow, wemb, bemb, wc, bc):
    return pl.pallas_call(
        _prologue1_body,
        out_shape=jax.ShapeDtypeStruct((N, H), _F32),
    )(x, coords, bcol, brow, wemb, bemb, wc, bc)


def _prologue2_body(h_ref, coords_ref, bcol_ref, brow_ref, watt_ref, batt_ref,
                    wvn_ref, bvn_ref, ht_ref):
    coords = coords_ref[...]
    h = h_ref[...]
    S = (bcol_ref[...] ==
         lax.broadcasted_iota(jnp.int32, (N, G), 1)).astype(_F32)
    St = (brow_ref[...] ==
          lax.broadcasted_iota(jnp.int32, (G, N), 0)).astype(_F32)
    logits = _dot(h, watt_ref[...]) + batt_ref[...]
    neg = jnp.float32(-1e30)
    lmax0 = jnp.max(jnp.where(S > 0, logits[:, 0:1], neg), axis=0,
                    keepdims=True)
    lmax1 = jnp.max(jnp.where(S > 0, logits[:, 1:2], neg), axis=0,
                    keepdims=True)
    lmax_b = jnp.concatenate([_dot_t(S, lmax0), _dot_t(S, lmax1)], axis=1)
    e = jnp.exp(logits - lmax_b)
    denom_b = _dot(S, _dot(St, e))
    w = e / jnp.maximum(denom_b, 1e-9)
    ds = []
    for v in range(2):
        vnp = _dot(St, w[:, v:v + 1] * coords)
        dv = coords - _dot(S, vnp)
        ds.append(jnp.sqrt(jnp.sum(dv * dv, axis=1, keepdims=True)))
    d = jnp.concatenate(ds, axis=1)
    hv = h + _silu(_dot(d, wvn_ref[...]) + bvn_ref[...])
    ht_ref[...] = jnp.concatenate(
        [hv, coords, jnp.zeros((N, D - H - 3), _F32)], axis=1)


def _prologue2(h, coords, bcol, brow, watt, batt, wvn, bvn):
    return pl.pallas_call(
        _prologue2_body,
        out_shape=jax.ShapeDtypeStruct((N, D), _F32),
    )(h, coords, bcol, brow, watt, batt, wvn, bvn)


def _sc_gather_call(ht, rowr, colr):
    mesh = plsc.VectorSubcoreMesh(core_axis_name="c", subcore_axis_name="s")

    @functools.partial(
        pl.kernel,
        out_type=(jax.ShapeDtypeStruct((E, D), _F32),
                  jax.ShapeDtypeStruct((E, D), _F32)),
        mesh=mesh,
    )
    def k(ht_hbm, row_hbm, col_hbm, grow_hbm, gcol_hbm):
        def body(ir_vmem, ic_vmem, orow_vmem, ocol_vmem):
            pltpu.sync_copy(ht_hbm.at[ir_vmem.at[0]], orow_vmem)
            pltpu.sync_copy(ht_hbm.at[ic_vmem.at[0]], ocol_vmem)

        pltpu.emit_pipeline(
            body,
            grid=(E // GW,),
            in_specs=[pl.BlockSpec((1, GW), lambda i: (0, i)),
                      pl.BlockSpec((1, GW), lambda i: (0, i))],
            out_specs=[pl.BlockSpec((GW, D), lambda i: (i, 0)),
                       pl.BlockSpec((GW, D), lambda i: (i, 0))],
            core_axis_name=("c", "s"),
            dimension_semantics=(pltpu.PARALLEL,),
        )(row_hbm, col_hbm, grow_hbm, gcol_hbm)

    return k(ht, rowr, colr)


def _sc_scatter_call(comb, rowr, zrows):
    mesh = plsc.VectorSubcoreMesh(core_axis_name="c", subcore_axis_name="s")

    @functools.partial(
        pl.kernel,
        out_type=jax.ShapeDtypeStruct((NCORE, N, D), _F32),
        mesh=mesh,
        scratch_types=[pltpu.VMEM_SHARED((N, D), _F32)],
    )
    def k(comb_hbm, row_hbm, z_hbm, out_hbm, acc_sh):
        c = lax.axis_index("c")
        s = lax.axis_index("s")

        @pl.when(s < NSTRIPE)
        def _():
            pltpu.sync_copy(z_hbm, acc_sh.at[pl.ds(s * RPS, RPS)])

        plsc.subcore_barrier()

        def body(cb_vmem, idx_vmem):
            pltpu.sync_copy(cb_vmem, acc_sh.at[idx_vmem.at[0]], add=True)

        pltpu.emit_pipeline(
            body,
            grid=(E // SW,),
            in_specs=[pl.BlockSpec((SW, D), lambda i: (i, 0)),
                      pl.BlockSpec((1, SW), lambda i: (0, i))],
            out_specs=[],
            core_axis_name=("c", "s"),
            dimension_semantics=(pltpu.PARALLEL,),
        )(comb_hbm, row_hbm)

        plsc.subcore_barrier()

        @pl.when(s < NSTRIPE)
        def _():
            pltpu.sync_copy(acc_sh.at[pl.ds(s * RPS, RPS)],
                            out_hbm.at[c].at[pl.ds(s * RPS, RPS)])

    return k(comb, rowr, zrows)


def _edge_body(grow_ref, gcol_ref, ea_ref, w1a_ref, w1b_ref, w1c_ref, w1d_ref,
               b1_ref, w2_ref, b2_ref, wc1_ref, bc1_ref, wc2_ref, bc2_ref,
               out_ref):
    grow = grow_ref[...]
    gcol = gcol_ref[...]
    rel = grow[:, H:H + 3] - gcol[:, H:H + 3]
    dist2 = jnp.sum(rel * rel, axis=1, keepdims=True)
    t1 = (_dot(grow.astype(_BF16), w1a_ref[...]) +
          _dot(gcol.astype(_BF16), w1b_ref[...]) +
          dist2 * w1c_ref[...] +
          _dot(ea_ref[...].astype(_BF16), w1d_ref[...]) + b1_ref[...])
    m = _silu(_dot(_silu(t1).astype(_BF16), w2_ref[...]) + b2_ref[...])
    t2 = _silu(_dot(m.astype(_BF16), wc1_ref[...]) + bc1_ref[...])
    cw = jnp.sum(t2 * wc2_ref[...], axis=1, keepdims=True) + bc2_ref[...]
    out_ref[...] = jnp.concatenate(
        [m, rel * cw, jnp.ones((BE, 1), _F32), jnp.zeros((BE, D - H - 4), _F32)],
        axis=1)


def _edge_call(grow, gcol, ea, wts):
    nblk = E // BE
    full = lambda shape: pl.BlockSpec(shape, lambda i: (0, 0))
    return pl.pallas_call(
        _edge_body,
        grid=(nblk,),
        in_specs=[
            pl.BlockSpec((BE, D), lambda i: (i, 0)),
            pl.BlockSpec((BE, D), lambda i: (i, 0)),
            pl.BlockSpec((BE, EF), lambda i: (i, 0)),
            full((D, H)), full((D, H)), full((1, H)), full((EF, H)),
            full((1, H)), full((H, H)), full((1, H)),
            full((H, H)), full((1, H)), full((1, H)), full((1, 1)),
        ],
        out_specs=pl.BlockSpec((BE, D), lambda i: (i, 0)),
        out_shape=jax.ShapeDtypeStruct((E, D), _F32),
    )(grow, gcol, ea, *wts)


def _node_body(acc_ref, ht_ref, wn1a_ref, wn1b_ref, bn1_ref, wn2_ref, bn2_ref,
               out_ref):
    asum = acc_ref[0] + acc_ref[1]
    macc = asum[:, 0:H]
    pacc = asum[:, H:H + 3]
    cnt = asum[:, H + 3:H + 4]
    h = ht_ref[:, 0:H]
    pos = ht_ref[:, H:H + 3]
    pos_new = pos + pacc / jnp.maximum(cnt, 1.0)
    u = _silu(_dot(h, wn1a_ref[...]) + _dot(macc, wn1b_ref[...]) + bn1_ref[...])
    h_new = h + _dot(u, wn2_ref[...]) + bn2_ref[...]
    out_ref[...] = jnp.concatenate(
        [h_new, pos_new, jnp.zeros((N, D - H - 3), _F32)], axis=1)


def _node_call(acc, ht, wts):
    return pl.pallas_call(
        _node_body,
        out_shape=jax.ShapeDtypeStruct((N, D), _F32),
    )(acc, ht, *wts)


def _epi_body(ht_ref, brow_ref, seq_ref, wseq_ref, bseq_ref, w1a_ref, w1b_ref,
              b1_ref, w2_ref, b2_ref, out_ref):
    St = (brow_ref[...] ==
          lax.broadcasted_iota(jnp.int32, (G, N), 0)).astype(_F32)
    h = ht_ref[:, 0:H]
    cnt = jnp.sum(St, axis=1, keepdims=True)
    gf = _dot(St, h) / jnp.maximum(cnt, 1.0)
    seq = jnp.maximum(_dot(seq_ref[...], wseq_ref[...]) + bseq_ref[...], 0.0)
    o1 = jnp.maximum(_dot(gf, w1a_ref[...]) + _dot(seq, w1b_ref[...]) +
                     b1_ref[...], 0.0)
    out_ref[...] = _dot(o1, w2_ref[...]) + b2_ref[...]


def _epi_call(ht, brow, seq_feat, wts):
    return pl.pallas_call(
        _epi_body,
        out_shape=jax.ShapeDtypeStruct((G, 2), _F32),
    )(ht, brow, seq_feat, *wts)


def kernel(x, coords, batch, edge_index, edge_attr, seq_feat, params):
    x = x.astype(_F32)
    coords = coords.astype(_F32)
    edge_attr = edge_attr.astype(_F32)
    bi = batch.astype(jnp.int32)
    row = edge_index[0].astype(jnp.int32)
    col = edge_index[1].astype(jnp.int32)
    bcol = bi.reshape(N, 1)
    brow = bi.reshape(1, N)
    rowr = row.reshape(1, E)
    colr = col.reshape(1, E)
    p = params

    r1 = lambda b: b.reshape(1, -1)
    h1 = _prologue1(x, coords, bcol, brow,
                    p["embedding"]["w"], r1(p["embedding"]["b"]),
                    p["color"]["w"].reshape(1, H), r1(p["color"]["b"]))
    ht = _prologue2(h1, coords, bcol, brow,
                          p["vn_att"]["w"], r1(p["vn_att"]["b"]),
                          p["vn_feat"]["w"], r1(p["vn_feat"]["b"]))

    zrows = jnp.zeros((RPS, D), _F32)
    for lp in p["layers"]:
        w1 = lp["edge1"]["w"]
        w1a = jnp.zeros((D, H), _F32).at[0:H].set(w1[0:H]).astype(_BF16)
        w1b = jnp.zeros((D, H), _F32).at[0:H].set(w1[H:2 * H]).astype(_BF16)
        ewts = (w1a, w1b, w1[2 * H:2 * H + 1],
                w1[2 * H + 1:].astype(_BF16), r1(lp["edge1"]["b"]),
                lp["edge2"]["w"].astype(_BF16), r1(lp["edge2"]["b"]),
                lp["coord1"]["w"].astype(_BF16), r1(lp["coord1"]["b"]),
                lp["coord2"]["w"].reshape(1, H), r1(lp["coord2"]["b"]))
        wn1 = lp["node1"]["w"]
        nwts = (wn1[0:H], wn1[H:], r1(lp["node1"]["b"]),
                lp["node2"]["w"], r1(lp["node2"]["b"]))

        grow, gcol = _sc_gather_call(ht, rowr, colr)
        comb = _edge_call(grow, gcol, edge_attr, ewts)
        acc = _sc_scatter_call(comb, rowr, zrows)
        ht = _node_call(acc, ht, nwts)

    wl1 = p["lin1"]["w"]
    out = _epi_call(ht, brow, seq_feat.reshape(G, SEQ),
                    (p["seq_fc"]["w"], r1(p["seq_fc"]["b"]),
                     wl1[0:H], wl1[H:], r1(p["lin1"]["b"]),
                     p["lin2"]["w"], r1(p["lin2"]["b"])))
    return out


# speedup vs baseline: 1.7346x; 1.7346x over previous
import functools

import jax
import jax.numpy as jnp
from jax import lax
from jax.experimental import pallas as pl
from jax.experimental.pallas import tpu as pltpu
from jax.experimental.pallas import tpu_sc as plsc

N = 10000
E = 320000
G = 64
H = 64
NF = 81
EF = 7
SEQ = 15 * 2560
D = 128
NSUB = 16
NCORE = 2
RPS = 1000
NSTRIPE = N // RPS
GW = 128
SW = 128
BE = 2560

_F32 = jnp.float32
_BF16 = jnp.bfloat16


def _silu(v):
    return v * (1.0 / (1.0 + jnp.exp(-v)))


def _dot(a, b):
    return jnp.dot(a, b, preferred_element_type=_F32)


def _dot_t(a, b):
    return lax.dot_general(a, b, (((1,), (1,)), ((), ())),
                           preferred_element_type=_F32)


def _prologue1_body(x_ref, coords_ref, bcol_ref, brow_ref, wemb_ref, bemb_ref,
                    wc_ref, bc_ref, h_ref):
    coords = coords_ref[...]
    S = (bcol_ref[...] ==
         lax.broadcasted_iota(jnp.int32, (N, G), 1)).astype(_F32)
    St = (brow_ref[...] ==
          lax.broadcasted_iota(jnp.int32, (G, N), 0)).astype(_F32)
    h = _dot(x_ref[...], wemb_ref[...]) + bemb_ref[...]
    csum = _dot(St, coords)
    cnt = jnp.sum(St, axis=1, keepdims=True)
    center = csum / jnp.maximum(cnt, 1.0)
    diff = coords - _dot(S, center)
    radius = jnp.sqrt(jnp.sum(diff * diff, axis=1, keepdims=True))
    h_ref[...] = h + _silu(radius * wc_ref[...] + bc_ref[...])


def _prologue1(x, coords, bcol, brow, wemb, bemb, wc, bc):
    return pl.pallas_call(
        _prologue1_body,
        out_shape=jax.ShapeDtypeStruct((N, H), _F32),
    )(x, coords, bcol, brow, wemb, bemb, wc, bc)


def _prologue2_body(h_ref, coords_ref, bcol_ref, brow_ref, watt_ref, batt_ref,
                    wvn_ref, bvn_ref, ht_ref):
    coords = coords_ref[...]
    h = h_ref[...]
    S = (bcol_ref[...] ==
         lax.broadcasted_iota(jnp.int32, (N, G), 1)).astype(_F32)
    St = (brow_ref[...] ==
          lax.broadcasted_iota(jnp.int32, (G, N), 0)).astype(_F32)
    logits = _dot(h, watt_ref[...]) + batt_ref[...]
    neg = jnp.float32(-1e30)
    lmax0 = jnp.max(jnp.where(S > 0, logits[:, 0:1], neg), axis=0,
                    keepdims=True)
    lmax1 = jnp.max(jnp.where(S > 0, logits[:, 1:2], neg), axis=0,
                    keepdims=True)
    lmax_b = jnp.concatenate([_dot_t(S, lmax0), _dot_t(S, lmax1)], axis=1)
    e = jnp.exp(logits - lmax_b)
    denom_b = _dot(S, _dot(St, e))
    w = e / jnp.maximum(denom_b, 1e-9)
    ds = []
    for v in range(2):
        vnp = _dot(St, w[:, v:v + 1] * coords)
        dv = coords - _dot(S, vnp)
        ds.append(jnp.sqrt(jnp.sum(dv * dv, axis=1, keepdims=True)))
    d = jnp.concatenate(ds, axis=1)
    hv = h + _silu(_dot(d, wvn_ref[...]) + bvn_ref[...])
    ht_ref[...] = jnp.concatenate(
        [hv, coords, jnp.zeros((N, D - H - 3), _F32)], axis=1)


def _prologue2(h, coords, bcol, brow, watt, batt, wvn, bvn):
    return pl.pallas_call(
        _prologue2_body,
        out_shape=jax.ShapeDtypeStruct((N, D), _F32),
    )(h, coords, bcol, brow, watt, batt, wvn, bvn)


def _sc_gather_call(ht, rowr, colr):
    mesh = plsc.VectorSubcoreMesh(core_axis_name="c", subcore_axis_name="s")

    @functools.partial(
        pl.kernel,
        out_type=(jax.ShapeDtypeStruct((E, D), _F32),
                  jax.ShapeDtypeStruct((E, D), _F32)),
        mesh=mesh,
        scratch_types=[pltpu.SemaphoreType.DMA, pltpu.SemaphoreType.DMA],
    )
    def k(ht_hbm, row_hbm, col_hbm, grow_hbm, gcol_hbm, sem1, sem2):
        def body(ir_vmem, ic_vmem, orow_vmem, ocol_vmem):
            cp1 = pltpu.async_copy(ht_hbm.at[ir_vmem.at[0]], orow_vmem, sem1)
            cp2 = pltpu.async_copy(ht_hbm.at[ic_vmem.at[0]], ocol_vmem, sem2)
            cp1.wait()
            cp2.wait()

        pltpu.emit_pipeline(
            body,
            grid=(E // GW,),
            in_specs=[pl.BlockSpec((1, GW), lambda i: (0, i)),
                      pl.BlockSpec((1, GW), lambda i: (0, i))],
            out_specs=[pl.BlockSpec((GW, D), lambda i: (i, 0)),
                       pl.BlockSpec((GW, D), lambda i: (i, 0))],
            core_axis_name=("c", "s"),
            dimension_semantics=(pltpu.PARALLEL,),
        )(row_hbm, col_hbm, grow_hbm, gcol_hbm)

    return k(ht, rowr, colr)


def _sc_scatter_call(comb, rowr, zrows):
    mesh = plsc.VectorSubcoreMesh(core_axis_name="c", subcore_axis_name="s")

    @functools.partial(
        pl.kernel,
        out_type=jax.ShapeDtypeStruct((NCORE, N, D), _F32),
        mesh=mesh,
        scratch_types=[pltpu.VMEM_SHARED((N, D), _F32)],
    )
    def k(comb_hbm, row_hbm, z_hbm, out_hbm, acc_sh):
        c = lax.axis_index("c")
        s = lax.axis_index("s")

        @pl.when(s < NSTRIPE)
        def _():
            pltpu.sync_copy(z_hbm, acc_sh.at[pl.ds(s * RPS, RPS)])

        plsc.subcore_barrier()

        def body(cb_vmem, idx_vmem):
            pltpu.sync_copy(cb_vmem, acc_sh.at[idx_vmem.at[0]], add=True)

        pltpu.emit_pipeline(
            body,
            grid=(E // SW,),
            in_specs=[pl.BlockSpec((SW, D), lambda i: (i, 0)),
                      pl.BlockSpec((1, SW), lambda i: (0, i))],
            out_specs=[],
            core_axis_name=("c", "s"),
            dimension_semantics=(pltpu.PARALLEL,),
        )(comb_hbm, row_hbm)

        plsc.subcore_barrier()

        @pl.when(s < NSTRIPE)
        def _():
            pltpu.sync_copy(acc_sh.at[pl.ds(s * RPS, RPS)],
                            out_hbm.at[c].at[pl.ds(s * RPS, RPS)])

    return k(comb, rowr, zrows)


def _edge_body(grow_ref, gcol_ref, ea_ref, w1a_ref, w1b_ref, w1c_ref, w1d_ref,
               b1_ref, w2_ref, b2_ref, wc1_ref, bc1_ref, wc2_ref, bc2_ref,
               out_ref):
    grow = grow_ref[...]
    gcol = gcol_ref[...]
    rel = grow[:, H:H + 3] - gcol[:, H:H + 3]
    dist2 = jnp.sum(rel * rel, axis=1, keepdims=True)
    t1 = (_dot(grow.astype(_BF16), w1a_ref[...]) +
          _dot(gcol.astype(_BF16), w1b_ref[...]) +
          dist2 * w1c_ref[...] +
          _dot(ea_ref[...].astype(_BF16), w1d_ref[...]) + b1_ref[...])
    m = _silu(_dot(_silu(t1).astype(_BF16), w2_ref[...]) + b2_ref[...])
    t2 = _silu(_dot(m.astype(_BF16), wc1_ref[...]) + bc1_ref[...])
    cw = jnp.sum(t2 * wc2_ref[...], axis=1, keepdims=True) + bc2_ref[...]
    out_ref[...] = jnp.concatenate(
        [m, rel * cw, jnp.ones((BE, 1), _F32), jnp.zeros((BE, D - H - 4), _F32)],
        axis=1)


def _edge_call(grow, gcol, ea, wts):
    nblk = E // BE
    full = lambda shape: pl.BlockSpec(shape, lambda i: (0, 0))
    return pl.pallas_call(
        _edge_body,
        grid=(nblk,),
        in_specs=[
            pl.BlockSpec((BE, D), lambda i: (i, 0)),
            pl.BlockSpec((BE, D), lambda i: (i, 0)),
            pl.BlockSpec((BE, EF), lambda i: (i, 0)),
            full((D, H)), full((D, H)), full((1, H)), full((EF, H)),
            full((1, H)), full((H, H)), full((1, H)),
            full((H, H)), full((1, H)), full((1, H)), full((1, 1)),
        ],
        out_specs=pl.BlockSpec((BE, D), lambda i: (i, 0)),
        out_shape=jax.ShapeDtypeStruct((E, D), _F32),
    )(grow, gcol, ea, *wts)


def _node_body(acc_ref, ht_ref, wn1a_ref, wn1b_ref, bn1_ref, wn2_ref, bn2_ref,
               out_ref):
    asum = acc_ref[0] + acc_ref[1]
    macc = asum[:, 0:H]
    pacc = asum[:, H:H + 3]
    cnt = asum[:, H + 3:H + 4]
    h = ht_ref[:, 0:H]
    pos = ht_ref[:, H:H + 3]
    pos_new = pos + pacc / jnp.maximum(cnt, 1.0)
    u = _silu(_dot(h, wn1a_ref[...]) + _dot(macc, wn1b_ref[...]) + bn1_ref[...])
    h_new = h + _dot(u, wn2_ref[...]) + bn2_ref[...]
    out_ref[...] = jnp.concatenate(
        [h_new, pos_new, jnp.zeros((N, D - H - 3), _F32)], axis=1)


def _node_call(acc, ht, wts):
    return pl.pallas_call(
        _node_body,
        out_shape=jax.ShapeDtypeStruct((N, D), _F32),
    )(acc, ht, *wts)


def _epi_body(ht_ref, brow_ref, seq_ref, wseq_ref, bseq_ref, w1a_ref, w1b_ref,
              b1_ref, w2_ref, b2_ref, out_ref):
    St = (brow_ref[...] ==
          lax.broadcasted_iota(jnp.int32, (G, N), 0)).astype(_F32)
    h = ht_ref[:, 0:H]
    cnt = jnp.sum(St, axis=1, keepdims=True)
    gf = _dot(St, h) / jnp.maximum(cnt, 1.0)
    seq = jnp.maximum(_dot(seq_ref[...], wseq_ref[...]) + bseq_ref[...], 0.0)
    o1 = jnp.maximum(_dot(gf, w1a_ref[...]) + _dot(seq, w1b_ref[...]) +
                     b1_ref[...], 0.0)
    out_ref[...] = _dot(o1, w2_ref[...]) + b2_ref[...]


def _epi_call(ht, brow, seq_feat, wts):
    return pl.pallas_call(
        _epi_body,
        out_shape=jax.ShapeDtypeStruct((G, 2), _F32),
    )(ht, brow, seq_feat, *wts)


def kernel(x, coords, batch, edge_index, edge_attr, seq_feat, params):
    x = x.astype(_F32)
    coords = coords.astype(_F32)
    edge_attr = edge_attr.astype(_F32)
    bi = batch.astype(jnp.int32)
    row = edge_index[0].astype(jnp.int32)
    col = edge_index[1].astype(jnp.int32)
    bcol = bi.reshape(N, 1)
    brow = bi.reshape(1, N)
    rowr = row.reshape(1, E)
    colr = col.reshape(1, E)
    p = params

    r1 = lambda b: b.reshape(1, -1)
    h1 = _prologue1(x, coords, bcol, brow,
                    p["embedding"]["w"], r1(p["embedding"]["b"]),
                    p["color"]["w"].reshape(1, H), r1(p["color"]["b"]))
    ht = _prologue2(h1, coords, bcol, brow,
                          p["vn_att"]["w"], r1(p["vn_att"]["b"]),
                          p["vn_feat"]["w"], r1(p["vn_feat"]["b"]))

    zrows = jnp.zeros((RPS, D), _F32)
    for lp in p["layers"]:
        w1 = lp["edge1"]["w"]
        w1a = jnp.zeros((D, H), _F32).at[0:H].set(w1[0:H]).astype(_BF16)
        w1b = jnp.zeros((D, H), _F32).at[0:H].set(w1[H:2 * H]).astype(_BF16)
        ewts = (w1a, w1b, w1[2 * H:2 * H + 1],
                w1[2 * H + 1:].astype(_BF16), r1(lp["edge1"]["b"]),
                lp["edge2"]["w"].astype(_BF16), r1(lp["edge2"]["b"]),
                lp["coord1"]["w"].astype(_BF16), r1(lp["coord1"]["b"]),
                lp["coord2"]["w"].reshape(1, H), r1(lp["coord2"]["b"]))
        wn1 = lp["node1"]["w"]
        nwts = (wn1[0:H], wn1[H:], r1(lp["node1"]["b"]),
                lp["node2"]["w"], r1(lp["node2"]["b"]))

        grow, gcol = _sc_gather_call(ht, rowr, colr)
        comb = _edge_call(grow, gcol, edge_attr, ewts)
        acc = _sc_scatter_call(comb, rowr, zrows)
        ht = _node_call(acc, ht, nwts)

    wl1 = p["lin1"]["w"]
    out = _epi_call(ht, brow, seq_feat.reshape(G, SEQ),
                    (p["seq_fc"]["w"], r1(p["seq_fc"]["b"]),
                     wl1[0:H], wl1[H:], r1(p["lin1"]["b"]),
                     p["lin2"]["w"], r1(p["lin2"]["b"])))
    return out
